# baseline (speedup 1.0000x reference)
import jax
import jax.numpy as jnp
from jax import lax
from jax.experimental import pallas as pl
from jax.experimental.pallas import tpu as pltpu
from jax.experimental.pallas import tpu_sc as plsc

_BB = 4


def _pos_gather_sc(y_table, x_table, y_idx, x_idx, seq):
    n_rows, half = y_table.shape
    info = plsc.get_sparse_core_info()
    nc, ns, nl = info.num_cores, info.num_subcores, info.num_lanes
    nw = nc * ns
    per_w = seq // nw
    mesh = plsc.VectorSubcoreMesh(core_axis_name="c", subcore_axis_name="s")

    def body(yt_hbm, xt_hbm, yidx_hbm, xidx_hbm, yemb_hbm, xemb_hbm,
             yidx_v, xidx_v, yrows_v, xrows_v, sem_y, sem_x):
        wid = lax.axis_index("s") * nc + lax.axis_index("c")
        base = wid * per_w
        pltpu.sync_copy(yidx_hbm.at[pl.ds(base, per_w)], yidx_v)
        pltpu.sync_copy(xidx_hbm.at[pl.ds(base, per_w)], xidx_v)
        cp_y = pltpu.async_copy(yt_hbm.at[yidx_v], yrows_v, sem_y)
        cp_x = pltpu.async_copy(xt_hbm.at[xidx_v], xrows_v, sem_x)
        cp_y.wait()
        cp_x.wait()
        pltpu.sync_copy(yrows_v, yemb_hbm.at[pl.ds(base, per_w)])
        pltpu.sync_copy(xrows_v, xemb_hbm.at[pl.ds(base, per_w)])

    f = pl.kernel(
        body,
        mesh=mesh,
        out_type=[
            jax.ShapeDtypeStruct((seq, half), jnp.float32),
            jax.ShapeDtypeStruct((seq, half), jnp.float32),
        ],
        scratch_types=[
            pltpu.VMEM((per_w,), jnp.int32),
            pltpu.VMEM((per_w,), jnp.int32),
            pltpu.VMEM((per_w, half), jnp.float32),
            pltpu.VMEM((per_w, half), jnp.float32),
            pltpu.SemaphoreType.DMA,
            pltpu.SemaphoreType.DMA,
        ],
    )
    return f(y_table, x_table, y_idx, x_idx)


def _add_body(x_ref, ye_ref, xe_ref, o_ref):
    half = ye_ref.shape[1]
    ye = ye_ref[...]
    xe = xe_ref[...]
    for i in range(o_ref.shape[0]):
        o_ref[i, :, :half] = x_ref[i, :, :half] + ye
        o_ref[i, :, half:] = x_ref[i, :, half:] + xe


def kernel(x, y_table, x_table, h, w):
    B, seq, D = x.shape
    half = D // 2
    p = jnp.arange(seq, dtype=jnp.int32)
    r = p // jnp.asarray(w, jnp.int32)
    y_idx = jnp.minimum(r, jnp.asarray(h, jnp.int32) - 1)
    x_idx = p - r * jnp.asarray(w, jnp.int32)
    y_emb, x_emb = _pos_gather_sc(y_table, x_table, y_idx, x_idx, seq)

    return pl.pallas_call(
        _add_body,
        grid=(B // _BB,),
        in_specs=[
            pl.BlockSpec((_BB, seq, D), lambda b: (b, 0, 0)),
            pl.BlockSpec((seq, half), lambda b: (0, 0)),
            pl.BlockSpec((seq, half), lambda b: (0, 0)),
        ],
        out_specs=pl.BlockSpec((_BB, seq, D), lambda b: (b, 0, 0)),
        out_shape=jax.ShapeDtypeStruct((B, seq, D), x.dtype),
    )(x, y_emb, x_emb)


# speedup vs baseline: 1.2074x; 1.2074x over previous
import jax
import jax.numpy as jnp
from jax import lax
from jax.experimental import pallas as pl
from jax.experimental.pallas import tpu as pltpu

_BB = 4


def _body(hw_ref, x_ref, yt_ref, xt_ref, o_ref, pos_ref):
    seq = pos_ref.shape[0]
    n_rows = yt_ref.shape[0]
    half = yt_ref.shape[1]

    @pl.when(pl.program_id(0) == 0)
    def _compute_pos():
        h = hw_ref[0]
        w = hw_ref[1]
        p = lax.broadcasted_iota(jnp.int32, (seq, n_rows), 0)
        j = lax.broadcasted_iota(jnp.int32, (seq, n_rows), 1)
        inv_w = 1.0 / w.astype(jnp.float32)
        r = jnp.floor(p.astype(jnp.float32) * inv_w + 0.0009765625)
        r = r.astype(jnp.int32)
        y_idx = jnp.minimum(r, h - 1)
        x_idx = p - w * r
        oh_y = (y_idx == j).astype(jnp.float32)
        oh_x = (x_idx == j).astype(jnp.float32)

        yt = yt_ref[...]
        xt = xt_ref[...]
        yt_hi = yt.astype(jnp.bfloat16).astype(jnp.float32)
        xt_hi = xt.astype(jnp.bfloat16).astype(jnp.float32)
        z = jnp.zeros((n_rows, half), jnp.float32)
        bd = jnp.concatenate(
            [
                jnp.concatenate([yt_hi, z], axis=1),
                jnp.concatenate([yt - yt_hi, z], axis=1),
                jnp.concatenate([z, xt_hi], axis=1),
                jnp.concatenate([z, xt - xt_hi], axis=1),
            ],
            axis=0,
        )
        oh4 = jnp.concatenate([oh_y, oh_y, oh_x, oh_x], axis=1)
        pos_ref[...] = jnp.dot(oh4, bd, preferred_element_type=jnp.float32)

    pos = pos_ref[...]
    for i in range(o_ref.shape[0]):
        o_ref[i] = x_ref[i] + pos


def kernel(x, y_table, x_table, h, w):
    B, seq, D = x.shape
    hw = jnp.array([h, w], dtype=jnp.int32)

    grid_spec = pltpu.PrefetchScalarGridSpec(
        num_scalar_prefetch=1,
        grid=(B // _BB,),
        in_specs=[
            pl.BlockSpec((_BB, seq, D), lambda b, hw_ref: (b, 0, 0)),
            pl.BlockSpec(y_table.shape, lambda b, hw_ref: (0, 0)),
            pl.BlockSpec(x_table.shape, lambda b, hw_ref: (0, 0)),
        ],
        out_specs=pl.BlockSpec((_BB, seq, D), lambda b, hw_ref: (b, 0, 0)),
        scratch_shapes=[pltpu.VMEM((seq, D), jnp.float32)],
    )
    return pl.pallas_call(
        _body,
        grid_spec=grid_spec,
        out_shape=jax.ShapeDtypeStruct((B, seq, D), x.dtype),
    )(hw, x, y_table, x_table)
